# baseline (speedup 1.0000x reference)
import functools

import jax
import jax.numpy as jnp
from jax.experimental import pallas as pl
from jax.experimental.pallas import tpu as pltpu


def _fused_kernel(x_ref, w_ref, o_ref, *, inv_s, n_c):
    xb = x_ref[...]
    m = jnp.sum(xb, axis=1, keepdims=True) * inv_s

    lane = jax.lax.broadcasted_iota(jnp.int32, m.shape, 2)
    m_prev = jnp.where(lane == 0, 0.0, jnp.roll(m, 1, axis=2))
    m_next = jnp.where(lane == n_c - 1, 0.0, jnp.roll(m, -1, axis=2))
    z = w_ref[0] * m_prev + w_ref[1] * m + w_ref[2] * m_next
    scale = 1.0 / (1.0 + jnp.exp(-z))

    t = xb * scale
    sig = 1.0 / (1.0 + jnp.exp(-t))
    o_ref[...] = (xb * sig).astype(o_ref.dtype)


def kernel(x, conv_w):
    B, C, D, H, W = x.shape
    S = D * H * W
    dtype = x.dtype

    x3 = jnp.transpose(x, (0, 2, 3, 4, 1)).reshape(B, S, C)
    w = conv_w.astype(jnp.float32)

    bt = 4 if B % 4 == 0 and B >= 8 else 1
    out3 = pl.pallas_call(
        functools.partial(_fused_kernel, inv_s=1.0 / float(S), n_c=C),
        out_shape=jax.ShapeDtypeStruct((B, S, C), dtype),
        grid=(B // bt,),
        in_specs=[
            pl.BlockSpec((bt, S, C), lambda b: (b, 0, 0)),
            pl.BlockSpec(memory_space=pltpu.SMEM),
        ],
        out_specs=pl.BlockSpec((bt, S, C), lambda b: (b, 0, 0)),
        compiler_params=pltpu.CompilerParams(
            dimension_semantics=("parallel",),
            vmem_limit_bytes=48 * 1024 * 1024,
        ),
    )(x3, w)

    return jnp.transpose(out3.reshape(B, D, H, W, C), (0, 4, 1, 2, 3))


# speedup vs baseline: 1.0050x; 1.0050x over previous
import functools

import jax
import jax.numpy as jnp
from jax.experimental import pallas as pl
from jax.experimental.pallas import tpu as pltpu


def _fused_kernel(x_ref, w_ref, o_ref, *, inv_s, n_c):
    xb = x_ref[...]
    bt, s, _ = xb.shape
    ones_row = jnp.full((1, s), inv_s, jnp.float32)
    m = jnp.concatenate(
        [jax.lax.dot(ones_row, xb[i],
                     preferred_element_type=jnp.float32)[None]
         for i in range(bt)], axis=0)

    lane = jax.lax.broadcasted_iota(jnp.int32, m.shape, 2)
    m_prev = jnp.where(lane == 0, 0.0, jnp.roll(m, 1, axis=2))
    m_next = jnp.where(lane == n_c - 1, 0.0, jnp.roll(m, -1, axis=2))
    z = w_ref[0] * m_prev + w_ref[1] * m + w_ref[2] * m_next
    scale = 1.0 / (1.0 + jnp.exp(-z))

    t = xb * scale
    sig = 1.0 / (1.0 + jnp.exp(-t))
    o_ref[...] = (xb * sig).astype(o_ref.dtype)


def kernel(x, conv_w):
    B, C, D, H, W = x.shape
    S = D * H * W
    dtype = x.dtype

    x3 = jnp.transpose(x, (0, 2, 3, 4, 1)).reshape(B, S, C)
    w = conv_w.astype(jnp.float32)

    bt = 4 if B % 4 == 0 and B >= 8 else 1
    out3 = pl.pallas_call(
        functools.partial(_fused_kernel, inv_s=1.0 / float(S), n_c=C),
        out_shape=jax.ShapeDtypeStruct((B, S, C), dtype),
        grid=(B // bt,),
        in_specs=[
            pl.BlockSpec((bt, S, C), lambda b: (b, 0, 0)),
            pl.BlockSpec(memory_space=pltpu.SMEM),
        ],
        out_specs=pl.BlockSpec((bt, S, C), lambda b: (b, 0, 0)),
        compiler_params=pltpu.CompilerParams(
            dimension_semantics=("parallel",),
            vmem_limit_bytes=48 * 1024 * 1024,
        ),
    )(x3, w)

    return jnp.transpose(out3.reshape(B, D, H, W, C), (0, 4, 1, 2, 3))


# speedup vs baseline: 1.0114x; 1.0064x over previous
import functools

import jax
import jax.numpy as jnp
from jax.experimental import pallas as pl
from jax.experimental.pallas import tpu as pltpu


def _fused_kernel(x_ref, w_ref, o_ref, *, inv_s, n_c):
    xb = x_ref[...]
    bt, s, _ = xb.shape
    ones_row = jnp.full((1, s), inv_s, jnp.float32)
    m = jnp.concatenate(
        [jax.lax.dot(ones_row, xb[i],
                     preferred_element_type=jnp.float32)[None]
         for i in range(bt)], axis=0)

    lane = jax.lax.broadcasted_iota(jnp.int32, m.shape, 2)
    m_prev = jnp.where(lane == 0, 0.0, jnp.roll(m, 1, axis=2))
    m_next = jnp.where(lane == n_c - 1, 0.0, jnp.roll(m, -1, axis=2))
    z = w_ref[0] * m_prev + w_ref[1] * m + w_ref[2] * m_next
    scale = 1.0 / (1.0 + jnp.exp(-z))

    t = xb * (0.5 * scale)
    sig = 0.5 + 0.5 * jnp.tanh(t)
    o_ref[...] = (xb * sig).astype(o_ref.dtype)


def kernel(x, conv_w):
    B, C, D, H, W = x.shape
    S = D * H * W
    dtype = x.dtype

    x3 = jnp.transpose(x, (0, 2, 3, 4, 1)).reshape(B, S, C)
    w = conv_w.astype(jnp.float32)

    bt = 4 if B % 4 == 0 and B >= 8 else 1
    out3 = pl.pallas_call(
        functools.partial(_fused_kernel, inv_s=1.0 / float(S), n_c=C),
        out_shape=jax.ShapeDtypeStruct((B, S, C), dtype),
        grid=(B // bt,),
        in_specs=[
            pl.BlockSpec((bt, S, C), lambda b: (b, 0, 0)),
            pl.BlockSpec(memory_space=pltpu.SMEM),
        ],
        out_specs=pl.BlockSpec((bt, S, C), lambda b: (b, 0, 0)),
        compiler_params=pltpu.CompilerParams(
            dimension_semantics=("parallel",),
            vmem_limit_bytes=48 * 1024 * 1024,
        ),
    )(x3, w)

    return jnp.transpose(out3.reshape(B, D, H, W, C), (0, 4, 1, 2, 3))
